# Initial kernel scaffold; baseline (speedup 1.0000x reference)
import jax
import jax.numpy as jnp
from jax.experimental import pallas as pl


def kernel(x, edge_index, W0, b0, g0, be0, W1, b1, g1, be1, W2, b2, g2, be2):
    raise NotImplementedError("write your pallas kernel here")


# speedup vs baseline: 4.4934x; 4.4934x over previous
import functools

import jax
import jax.numpy as jnp
from jax import lax
from jax.experimental import pallas as pl
from jax.experimental.pallas import tpu as pltpu
from jax.experimental.pallas import tpu_sc as plsc

N = 10000
D = 256
H = 128
NSC = 16
B = 128
NB = 80
EPW = NB * B
ACC_ROWS = 10016
R0 = 624
LZ = 656
LO = 640
TRASH = N
RB = 256
NRB = 40

_mesh = plsc.VectorSubcoreMesh(core_axis_name="c", subcore_axis_name="s")


@functools.partial(
    pl.kernel,
    out_type=jax.ShapeDtypeStruct((2 * N, H), jnp.float32),
    mesh=_mesh,
    scratch_types=[
        pltpu.VMEM_SHARED((ACC_ROWS, H), jnp.float32),
        pltpu.VMEM((NB, B), jnp.int32),
        pltpu.VMEM((NB, B), jnp.int32),
        pltpu.VMEM((B, H), jnp.float32),
    ],
)
def _sc_segsum(g2, srcp2, dstp, zrows, s_out, acc, srcv, dstv, buf):
    c = lax.axis_index("c")
    s = lax.axis_index("s")
    w = c * NSC + s
    pltpu.sync_copy(srcp2.at[w], srcv)
    pltpu.sync_copy(dstp.at[s], dstv)

    @pl.when(s < NSC - 1)
    def _():
        pltpu.sync_copy(zrows.at[pl.ds(0, R0)], acc.at[pl.ds(s * R0, R0)])

    @pl.when(s == NSC - 1)
    def _():
        pltpu.sync_copy(zrows, acc.at[pl.ds((NSC - 1) * R0, LZ)])

    plsc.subcore_barrier()

    @pl.loop(0, NB)
    def _(j):
        pltpu.sync_copy(g2.at[srcv.at[j]], buf)
        pltpu.sync_copy(buf, acc.at[dstv.at[j]], add=True)

    plsc.subcore_barrier()

    @pl.when(s < NSC - 1)
    def _():
        pltpu.sync_copy(acc.at[pl.ds(s * R0, R0)],
                        s_out.at[pl.ds(c * N + s * R0, R0)])

    @pl.when(s == NSC - 1)
    def _():
        pltpu.sync_copy(acc.at[pl.ds((NSC - 1) * R0, LO)],
                        s_out.at[pl.ds(c * N + (NSC - 1) * R0, LO)])


def _tc_pre0(x_ref, degp_ref, w_ref, g_ref, dinv_ref):
    deg = degp_ref[0, :, 0] + 1.0
    dinv = lax.rsqrt(deg)
    g = lax.dot_general(x_ref[...], w_ref[...], (((1,), (1,)), ((), ())),
                        preferred_element_type=jnp.float32)
    g = g * dinv[:, None]
    g_ref[0] = g[:, :H]
    g_ref[1] = g[:, H:]
    dinv_ref[...] = dinv[:, None]


def _tc_post(s_ref, g_ref, dinv_ref, b_ref, t_ref, ps_ref, pq_ref):
    i = pl.program_id(0)
    sv = jnp.concatenate([s_ref[0], s_ref[1]], axis=1)
    gv = jnp.concatenate([g_ref[0], g_ref[1]], axis=1)
    t = (sv + gv) * dinv_ref[...] + b_ref[...]
    rows = lax.broadcasted_iota(jnp.int32, (RB, 1), 0) + i * RB
    t = jnp.where(rows < N, t, 0.0)
    t_ref[...] = t
    ps_ref[...] = jnp.sum(t, axis=0, keepdims=True)[None]
    pq_ref[...] = jnp.sum(t * t, axis=0, keepdims=True)[None]


def _bn_coeffs(ps_ref, pq_ref, gam_ref, bet_ref):
    mu = jnp.sum(ps_ref[...], axis=0) / N
    ex2 = jnp.sum(pq_ref[...], axis=0) / N
    var = ex2 - mu * mu
    a = gam_ref[...] * lax.rsqrt(var + 1e-5)
    return a, bet_ref[...] - mu * a


def _tc_fuse(t_ref, ps_ref, pq_ref, gam_ref, bet_ref, dinv_ref, w_ref, g_ref):
    a, cshift = _bn_coeffs(ps_ref, pq_ref, gam_ref, bet_ref)
    z = t_ref[...] * a + cshift
    z = jnp.where(z >= 0, z, 0.01 * z)
    g = lax.dot_general(z, w_ref[...], (((1,), (1,)), ((), ())),
                        preferred_element_type=jnp.float32)
    g = g * dinv_ref[...]
    g_ref[0] = g[:, :H]
    g_ref[1] = g[:, H:]


def _tc_final(t_ref, ps_ref, pq_ref, gam_ref, bet_ref, o_ref):
    a, cshift = _bn_coeffs(ps_ref, pq_ref, gam_ref, bet_ref)
    o_ref[...] = t_ref[...] * a + cshift


_f32 = jnp.float32

_pre0_call = pl.pallas_call(
    _tc_pre0,
    grid=(NRB,),
    in_specs=[
        pl.BlockSpec((RB, D), lambda i: (i, 0)),
        pl.BlockSpec((2, RB, H), lambda i: (0, i, 0)),
        pl.BlockSpec((D, D), lambda i: (0, 0)),
    ],
    out_specs=[
        pl.BlockSpec((2, RB, H), lambda i: (0, i, 0)),
        pl.BlockSpec((RB, 1), lambda i: (i, 0)),
    ],
    out_shape=[
        jax.ShapeDtypeStruct((2, N, H), _f32),
        jax.ShapeDtypeStruct((N, 1), _f32),
    ],
)

_post_call = pl.pallas_call(
    _tc_post,
    grid=(NRB,),
    in_specs=[
        pl.BlockSpec((2, RB, H), lambda i: (0, i, 0)),
        pl.BlockSpec((2, RB, H), lambda i: (0, i, 0)),
        pl.BlockSpec((RB, 1), lambda i: (i, 0)),
        pl.BlockSpec((1, D), lambda i: (0, 0)),
    ],
    out_specs=[
        pl.BlockSpec((RB, D), lambda i: (i, 0)),
        pl.BlockSpec((1, 1, D), lambda i: (i, 0, 0)),
        pl.BlockSpec((1, 1, D), lambda i: (i, 0, 0)),
    ],
    out_shape=[
        jax.ShapeDtypeStruct((N, D), _f32),
        jax.ShapeDtypeStruct((NRB, 1, D), _f32),
        jax.ShapeDtypeStruct((NRB, 1, D), _f32),
    ],
)

_fuse_call = pl.pallas_call(
    _tc_fuse,
    grid=(NRB,),
    in_specs=[
        pl.BlockSpec((RB, D), lambda i: (i, 0)),
        pl.BlockSpec((NRB, 1, D), lambda i: (0, 0, 0)),
        pl.BlockSpec((NRB, 1, D), lambda i: (0, 0, 0)),
        pl.BlockSpec((1, D), lambda i: (0, 0)),
        pl.BlockSpec((1, D), lambda i: (0, 0)),
        pl.BlockSpec((RB, 1), lambda i: (i, 0)),
        pl.BlockSpec((D, D), lambda i: (0, 0)),
    ],
    out_specs=[pl.BlockSpec((2, RB, H), lambda i: (0, i, 0))],
    out_shape=[jax.ShapeDtypeStruct((2, N, H), _f32)],
)

_final_call = pl.pallas_call(
    _tc_final,
    grid=(NRB,),
    in_specs=[
        pl.BlockSpec((RB, D), lambda i: (i, 0)),
        pl.BlockSpec((NRB, 1, D), lambda i: (0, 0, 0)),
        pl.BlockSpec((NRB, 1, D), lambda i: (0, 0, 0)),
        pl.BlockSpec((1, D), lambda i: (0, 0)),
        pl.BlockSpec((1, D), lambda i: (0, 0)),
    ],
    out_specs=pl.BlockSpec((RB, D), lambda i: (i, 0)),
    out_shape=jax.ShapeDtypeStruct((N, D), _f32),
)


def kernel(x, edge_index, W0, b0, g0, be0, W1, b1, g1, be1, W2, b2, g2, be2):
    src = edge_index[0].astype(jnp.int32)
    dst = edge_index[1].astype(jnp.int32)
    e = src.shape[0]
    padn = NSC * EPW - e
    srcp = jnp.concatenate([src, jnp.zeros((padn,), jnp.int32)])
    srcp = srcp.reshape(NSC, NB, B)
    dstp = jnp.concatenate([dst, jnp.full((padn,), TRASH, jnp.int32)])
    dstp = dstp.reshape(NSC, NB, B)
    srcp2 = jnp.concatenate([srcp, srcp + N], axis=0)

    zrows = jnp.zeros((LZ, H), _f32)

    degp = _sc_segsum(jnp.ones((2 * N, H), _f32), srcp2, dstp, zrows).reshape(2, N, H)

    ga, dinv = _pre0_call(x, degp, W0)
    sa = _sc_segsum(ga.reshape(2 * N, H), srcp2, dstp, zrows).reshape(2, N, H)
    t, ps, pq = _post_call(sa, ga, dinv, b0.reshape(1, D))

    ga = _fuse_call(t, ps, pq, g0.reshape(1, D), be0.reshape(1, D), dinv, W1)[0]
    sa = _sc_segsum(ga.reshape(2 * N, H), srcp2, dstp, zrows).reshape(2, N, H)
    t, ps, pq = _post_call(sa, ga, dinv, b1.reshape(1, D))

    ga = _fuse_call(t, ps, pq, g1.reshape(1, D), be1.reshape(1, D), dinv, W2)[0]
    sa = _sc_segsum(ga.reshape(2 * N, H), srcp2, dstp, zrows).reshape(2, N, H)
    t, ps, pq = _post_call(sa, ga, dinv, b2.reshape(1, D))

    return _final_call(t, ps, pq, g2.reshape(1, D), be2.reshape(1, D))


# speedup vs baseline: 6.5508x; 1.4579x over previous
import functools

import jax
import jax.numpy as jnp
from jax import lax
from jax.experimental import pallas as pl
from jax.experimental.pallas import tpu as pltpu
from jax.experimental.pallas import tpu_sc as plsc

N = 10000
D = 256
H = 128
NSC = 16
B = 128
NB = 80
EPW = NB * B
ACC_ROWS = 10016
R0 = 624
LZ = 656
LO = 640
TRASH = N
CH = 16
NCH = NB // CH
RB = 256
NRB = 40

_mesh = plsc.VectorSubcoreMesh(core_axis_name="c", subcore_axis_name="s")


@functools.partial(
    pl.kernel,
    out_type=jax.ShapeDtypeStruct((2 * N, H), jnp.float32),
    mesh=_mesh,
    scratch_types=[
        pltpu.VMEM_SHARED((ACC_ROWS, H), jnp.float32),
        pltpu.VMEM((CH, B), jnp.int32),
        pltpu.VMEM((CH, B), jnp.int32),
        pltpu.VMEM((B, H), jnp.float32),
        pltpu.VMEM((B, H), jnp.float32),
        pltpu.SemaphoreType.DMA,
        pltpu.SemaphoreType.DMA,
    ],
)
def _sc_segsum(g2, srcp2, dstp2, zrows, s_out, acc, srcv, dstv, buf0, buf1,
               sem0, sem1):
    c = lax.axis_index("c")
    s = lax.axis_index("s")
    w = c * NSC + s

    @pl.when(s < NSC - 1)
    def _():
        pltpu.sync_copy(zrows.at[pl.ds(0, R0)], acc.at[pl.ds(s * R0, R0)])

    @pl.when(s == NSC - 1)
    def _():
        pltpu.sync_copy(zrows, acc.at[pl.ds((NSC - 1) * R0, LZ)])

    plsc.subcore_barrier()

    @pl.loop(0, NCH)
    def _(k):
        pltpu.sync_copy(srcp2.at[w * NCH + k], srcv)
        pltpu.sync_copy(dstp2.at[s * NCH + k], dstv)
        pltpu.async_copy(g2.at[srcv.at[0]], buf0, sem0)
        pltpu.async_copy(g2.at[srcv.at[1]], buf1, sem1)

        @pl.loop(0, CH, step=2)
        def _(i):
            pltpu.make_async_copy(g2.at[srcv.at[i]], buf0, sem0).wait()
            pltpu.sync_copy(buf0, acc.at[dstv.at[i]], add=True)

            @pl.when(i + 2 < CH)
            def _():
                pltpu.async_copy(g2.at[srcv.at[i + 2]], buf0, sem0)

            pltpu.make_async_copy(g2.at[srcv.at[i + 1]], buf1, sem1).wait()
            pltpu.sync_copy(buf1, acc.at[dstv.at[i + 1]], add=True)

            @pl.when(i + 3 < CH)
            def _():
                pltpu.async_copy(g2.at[srcv.at[i + 3]], buf1, sem1)

    plsc.subcore_barrier()

    @pl.when(s < NSC - 1)
    def _():
        pltpu.sync_copy(acc.at[pl.ds(s * R0, R0)],
                        s_out.at[pl.ds(c * N + s * R0, R0)])

    @pl.when(s == NSC - 1)
    def _():
        pltpu.sync_copy(acc.at[pl.ds((NSC - 1) * R0, LO)],
                        s_out.at[pl.ds(c * N + (NSC - 1) * R0, LO)])


@functools.partial(
    pl.kernel,
    out_type=jax.ShapeDtypeStruct((2 * N, H), jnp.float32),
    mesh=_mesh,
    scratch_types=[
        pltpu.VMEM_SHARED((ACC_ROWS, H), jnp.float32),
        pltpu.VMEM((NB, B), jnp.int32),
        pltpu.VMEM((B, H), jnp.float32),
    ],
)
def _sc_deg(dstp, ones_hbm, zrows, deg_out, acc, dstv, onesv):
    c = lax.axis_index("c")
    s = lax.axis_index("s")
    pltpu.sync_copy(dstp.at[s], dstv)
    pltpu.sync_copy(ones_hbm, onesv)

    @pl.when(s < NSC - 1)
    def _():
        pltpu.sync_copy(zrows.at[pl.ds(0, R0)], acc.at[pl.ds(s * R0, R0)])

    @pl.when(s == NSC - 1)
    def _():
        pltpu.sync_copy(zrows, acc.at[pl.ds((NSC - 1) * R0, LZ)])

    plsc.subcore_barrier()

    @pl.loop(0, NB)
    def _(j):
        @pl.when(j % 2 == c)
        def _():
            pltpu.sync_copy(onesv, acc.at[dstv.at[j]], add=True)

    plsc.subcore_barrier()

    @pl.when(s < NSC - 1)
    def _():
        pltpu.sync_copy(acc.at[pl.ds(s * R0, R0)],
                        deg_out.at[pl.ds(c * N + s * R0, R0)])

    @pl.when(s == NSC - 1)
    def _():
        pltpu.sync_copy(acc.at[pl.ds((NSC - 1) * R0, LO)],
                        deg_out.at[pl.ds(c * N + (NSC - 1) * R0, LO)])


def _tc_pre0(x_ref, degp_ref, w_ref, g_ref, dinv_ref):
    deg = degp_ref[0, :, 0] + degp_ref[1, :, 0] + 1.0
    dinv = lax.rsqrt(deg)
    g = lax.dot_general(x_ref[...], w_ref[...], (((1,), (1,)), ((), ())),
                        preferred_element_type=jnp.float32)
    g = g * dinv[:, None]
    g_ref[0] = g[:, :H]
    g_ref[1] = g[:, H:]
    dinv_ref[...] = dinv[:, None]


def _tc_post(s_ref, g_ref, dinv_ref, b_ref, t_ref, ps_ref, pq_ref):
    i = pl.program_id(0)
    sv = jnp.concatenate([s_ref[0], s_ref[1]], axis=1)
    gv = jnp.concatenate([g_ref[0], g_ref[1]], axis=1)
    t = (sv + gv) * dinv_ref[...] + b_ref[...]
    rows = lax.broadcasted_iota(jnp.int32, (RB, 1), 0) + i * RB
    t = jnp.where(rows < N, t, 0.0)
    t_ref[...] = t
    ps_ref[...] = jnp.sum(t, axis=0, keepdims=True)[None]
    pq_ref[...] = jnp.sum(t * t, axis=0, keepdims=True)[None]


def _bn_coeffs(ps_ref, pq_ref, gam_ref, bet_ref):
    mu = jnp.sum(ps_ref[...], axis=0) / N
    ex2 = jnp.sum(pq_ref[...], axis=0) / N
    var = ex2 - mu * mu
    a = gam_ref[...] * lax.rsqrt(var + 1e-5)
    return a, bet_ref[...] - mu * a


def _tc_fuse(t_ref, ps_ref, pq_ref, gam_ref, bet_ref, dinv_ref, w_ref, g_ref):
    a, cshift = _bn_coeffs(ps_ref, pq_ref, gam_ref, bet_ref)
    z = t_ref[...] * a + cshift
    z = jnp.where(z >= 0, z, 0.01 * z)
    g = lax.dot_general(z, w_ref[...], (((1,), (1,)), ((), ())),
                        preferred_element_type=jnp.float32)
    g = g * dinv_ref[...]
    g_ref[0] = g[:, :H]
    g_ref[1] = g[:, H:]


def _tc_final(t_ref, ps_ref, pq_ref, gam_ref, bet_ref, o_ref):
    a, cshift = _bn_coeffs(ps_ref, pq_ref, gam_ref, bet_ref)
    o_ref[...] = t_ref[...] * a + cshift


_f32 = jnp.float32

_pre0_call = pl.pallas_call(
    _tc_pre0,
    grid=(NRB,),
    in_specs=[
        pl.BlockSpec((RB, D), lambda i: (i, 0)),
        pl.BlockSpec((2, RB, H), lambda i: (0, i, 0)),
        pl.BlockSpec((D, D), lambda i: (0, 0)),
    ],
    out_specs=[
        pl.BlockSpec((2, RB, H), lambda i: (0, i, 0)),
        pl.BlockSpec((RB, 1), lambda i: (i, 0)),
    ],
    out_shape=[
        jax.ShapeDtypeStruct((2, N, H), _f32),
        jax.ShapeDtypeStruct((N, 1), _f32),
    ],
)

_post_call = pl.pallas_call(
    _tc_post,
    grid=(NRB,),
    in_specs=[
        pl.BlockSpec((2, RB, H), lambda i: (0, i, 0)),
        pl.BlockSpec((2, RB, H), lambda i: (0, i, 0)),
        pl.BlockSpec((RB, 1), lambda i: (i, 0)),
        pl.BlockSpec((1, D), lambda i: (0, 0)),
    ],
    out_specs=[
        pl.BlockSpec((RB, D), lambda i: (i, 0)),
        pl.BlockSpec((1, 1, D), lambda i: (i, 0, 0)),
        pl.BlockSpec((1, 1, D), lambda i: (i, 0, 0)),
    ],
    out_shape=[
        jax.ShapeDtypeStruct((N, D), _f32),
        jax.ShapeDtypeStruct((NRB, 1, D), _f32),
        jax.ShapeDtypeStruct((NRB, 1, D), _f32),
    ],
)

_fuse_call = pl.pallas_call(
    _tc_fuse,
    grid=(NRB,),
    in_specs=[
        pl.BlockSpec((RB, D), lambda i: (i, 0)),
        pl.BlockSpec((NRB, 1, D), lambda i: (0, 0, 0)),
        pl.BlockSpec((NRB, 1, D), lambda i: (0, 0, 0)),
        pl.BlockSpec((1, D), lambda i: (0, 0)),
        pl.BlockSpec((1, D), lambda i: (0, 0)),
        pl.BlockSpec((RB, 1), lambda i: (i, 0)),
        pl.BlockSpec((D, D), lambda i: (0, 0)),
    ],
    out_specs=[pl.BlockSpec((2, RB, H), lambda i: (0, i, 0))],
    out_shape=[jax.ShapeDtypeStruct((2, N, H), _f32)],
)

_final_call = pl.pallas_call(
    _tc_final,
    grid=(NRB,),
    in_specs=[
        pl.BlockSpec((RB, D), lambda i: (i, 0)),
        pl.BlockSpec((NRB, 1, D), lambda i: (0, 0, 0)),
        pl.BlockSpec((NRB, 1, D), lambda i: (0, 0, 0)),
        pl.BlockSpec((1, D), lambda i: (0, 0)),
        pl.BlockSpec((1, D), lambda i: (0, 0)),
    ],
    out_specs=pl.BlockSpec((RB, D), lambda i: (i, 0)),
    out_shape=jax.ShapeDtypeStruct((N, D), _f32),
)


def kernel(x, edge_index, W0, b0, g0, be0, W1, b1, g1, be1, W2, b2, g2, be2):
    src = edge_index[0].astype(jnp.int32)
    dst = edge_index[1].astype(jnp.int32)
    e = src.shape[0]
    padn = NSC * EPW - e
    srcp = jnp.concatenate([src, jnp.zeros((padn,), jnp.int32)])
    srcp = srcp.reshape(NSC, NB, B)
    dstp = jnp.concatenate([dst, jnp.full((padn,), TRASH, jnp.int32)])
    dstp = dstp.reshape(NSC, NB, B)
    srcp2 = jnp.concatenate([srcp, srcp + N], axis=0)
    srcp2c = srcp2.reshape(2 * NSC * NCH, CH, B)
    dstp2c = dstp.reshape(NSC * NCH, CH, B)

    zrows = jnp.zeros((LZ, H), _f32)

    degp = _sc_deg(dstp, jnp.ones((B, H), _f32), zrows).reshape(2, N, H)

    ga, dinv = _pre0_call(x, degp, W0)
    sa = _sc_segsum(ga.reshape(2 * N, H), srcp2c, dstp2c, zrows).reshape(2, N, H)
    t, ps, pq = _post_call(sa, ga, dinv, b0.reshape(1, D))

    ga = _fuse_call(t, ps, pq, g0.reshape(1, D), be0.reshape(1, D), dinv, W1)[0]
    sa = _sc_segsum(ga.reshape(2 * N, H), srcp2c, dstp2c, zrows).reshape(2, N, H)
    t, ps, pq = _post_call(sa, ga, dinv, b1.reshape(1, D))

    ga = _fuse_call(t, ps, pq, g1.reshape(1, D), be1.reshape(1, D), dinv, W2)[0]
    sa = _sc_segsum(ga.reshape(2 * N, H), srcp2c, dstp2c, zrows).reshape(2, N, H)
    t, ps, pq = _post_call(sa, ga, dinv, b2.reshape(1, D))

    return _final_call(t, ps, pq, g2.reshape(1, D), be2.reshape(1, D))


# speedup vs baseline: 10.2309x; 1.5618x over previous
import functools

import jax
import jax.numpy as jnp
from jax import lax
from jax.experimental import pallas as pl
from jax.experimental.pallas import tpu as pltpu
from jax.experimental.pallas import tpu_sc as plsc

N = 10000
D = 256
H = 128
NSC = 16
B = 128
NB = 80
EPW = NB * B
ACC_ROWS = 10016
R0 = 624
LZ = 656
LO = 640
TRASH = N
BS = 120
NBAT = 84
CH = 6
NCHUNK = NBAT // CH
RB = 256
NRB = 40

_mesh = plsc.VectorSubcoreMesh(core_axis_name="c", subcore_axis_name="s")


@functools.partial(
    pl.kernel,
    out_type=jax.ShapeDtypeStruct((2 * N, H), jnp.float32),
    mesh=_mesh,
    scratch_types=[
        pltpu.VMEM_SHARED((ACC_ROWS, H), jnp.float32),
        pltpu.VMEM((CH, BS), jnp.int32),
        pltpu.VMEM((CH, BS), jnp.int32),
        pltpu.VMEM((CH, BS), jnp.int32),
        pltpu.VMEM((CH, BS), jnp.int32),
        pltpu.VMEM((BS, H), jnp.float32),
        pltpu.VMEM((BS, H), jnp.float32),
        pltpu.VMEM((BS, H), jnp.float32),
        pltpu.SemaphoreType.DMA,
        pltpu.SemaphoreType.DMA,
        pltpu.SemaphoreType.DMA,
        pltpu.SemaphoreType.DMA,
        pltpu.SemaphoreType.DMA,
        pltpu.SemaphoreType.DMA,
        pltpu.SemaphoreType.DMA,
    ],
)
def _sc_segsum(g2, srcp2, dstp2, zrows, s_out, acc, sv0, dv0, sv1, dv1,
               buf0, buf1, buf2, sg0, sg1, sg2, ss0, ss1, ss2, si):
    c = lax.axis_index("c")
    s = lax.axis_index("s")
    w = c * NSC + s
    bufs = (buf0, buf1, buf2)
    sg = (sg0, sg1, sg2)
    ss = (ss0, ss1, ss2)
    svs = (sv0, sv1)
    dvs = (dv0, dv1)

    pltpu.sync_copy(srcp2.at[w * NCHUNK], sv0)
    pltpu.sync_copy(dstp2.at[s * NCHUNK], dv0)

    @pl.when(s < NSC - 1)
    def _():
        pltpu.sync_copy(zrows.at[pl.ds(0, R0)], acc.at[pl.ds(s * R0, R0)])

    @pl.when(s == NSC - 1)
    def _():
        pltpu.sync_copy(zrows, acc.at[pl.ds((NSC - 1) * R0, LZ)])

    pltpu.async_copy(g2.at[sv0.at[0]], buf0, sg0)
    pltpu.async_copy(g2.at[sv0.at[1]], buf1, sg1)
    plsc.subcore_barrier()

    def _chunk(kk, p):
        sv, dv = svs[p], dvs[p]
        svq, dvq = svs[1 - p], dvs[1 - p]
        for t in range(CH):
            b = t % 3
            bn = (t + 2) % 3
            pltpu.make_async_copy(g2.at[sv.at[t]], bufs[b], sg[b]).wait()
            pltpu.async_copy(bufs[b], acc.at[dv.at[t]], ss[b], add=True)

            if t == 2:
                @pl.when(kk < NCHUNK - 1)
                def _():
                    pltpu.async_copy(srcp2.at[w * NCHUNK + kk + 1], svq, si)
                    pltpu.async_copy(dstp2.at[s * NCHUNK + kk + 1], dvq, si)

            def _wait_prev(t=t, bn=bn):
                row = t - 1 if t >= 1 else CH - 1
                dref = dv if t >= 1 else dvq
                pltpu.make_async_copy(bufs[bn], acc.at[dref.at[row]],
                                      ss[bn]).wait()

            if t < CH - 2:
                if t == 0:
                    @pl.when(kk > 0)
                    def _():
                        _wait_prev()
                else:
                    _wait_prev()
                pltpu.async_copy(g2.at[sv.at[t + 2]], bufs[bn], sg[bn])
            else:
                @pl.when(kk < NCHUNK - 1)
                def _():
                    _wait_prev()
                    if t == CH - 2:
                        pltpu.make_async_copy(
                            srcp2.at[w * NCHUNK + kk + 1], svq, si).wait()
                        pltpu.make_async_copy(
                            dstp2.at[s * NCHUNK + kk + 1], dvq, si).wait()
                    pltpu.async_copy(g2.at[svq.at[t + 2 - CH]], bufs[bn],
                                     sg[bn])

    @pl.loop(0, NCHUNK)
    def _(kk):
        @pl.when(kk % 2 == 0)
        def _():
            _chunk(kk, 0)

        @pl.when(kk % 2 == 1)
        def _():
            _chunk(kk, 1)

    lastp = (NCHUNK - 1) % 2
    for t in range(CH - 3, CH):
        b = t % 3
        pltpu.make_async_copy(bufs[b], acc.at[dvs[lastp].at[t]],
                              ss[b]).wait()
    plsc.subcore_barrier()

    @pl.when(s < NSC - 1)
    def _():
        pltpu.sync_copy(acc.at[pl.ds(s * R0, R0)],
                        s_out.at[pl.ds(c * N + s * R0, R0)])

    @pl.when(s == NSC - 1)
    def _():
        pltpu.sync_copy(acc.at[pl.ds((NSC - 1) * R0, LO)],
                        s_out.at[pl.ds(c * N + (NSC - 1) * R0, LO)])


@functools.partial(
    pl.kernel,
    out_type=jax.ShapeDtypeStruct((2 * N, H), jnp.float32),
    mesh=_mesh,
    scratch_types=[
        pltpu.VMEM_SHARED((ACC_ROWS, H), jnp.float32),
        pltpu.VMEM((NB, B), jnp.int32),
        pltpu.VMEM((B, H), jnp.float32),
    ],
)
def _sc_deg(dstp, ones_hbm, zrows, deg_out, acc, dstv, onesv):
    c = lax.axis_index("c")
    s = lax.axis_index("s")
    pltpu.sync_copy(dstp.at[s], dstv)
    pltpu.sync_copy(ones_hbm, onesv)

    @pl.when(s < NSC - 1)
    def _():
        pltpu.sync_copy(zrows.at[pl.ds(0, R0)], acc.at[pl.ds(s * R0, R0)])

    @pl.when(s == NSC - 1)
    def _():
        pltpu.sync_copy(zrows, acc.at[pl.ds((NSC - 1) * R0, LZ)])

    plsc.subcore_barrier()

    @pl.loop(0, NB)
    def _(j):
        @pl.when(j % 2 == c)
        def _():
            pltpu.sync_copy(onesv, acc.at[dstv.at[j]], add=True)

    plsc.subcore_barrier()

    @pl.when(s < NSC - 1)
    def _():
        pltpu.sync_copy(acc.at[pl.ds(s * R0, R0)],
                        deg_out.at[pl.ds(c * N + s * R0, R0)])

    @pl.when(s == NSC - 1)
    def _():
        pltpu.sync_copy(acc.at[pl.ds((NSC - 1) * R0, LO)],
                        deg_out.at[pl.ds(c * N + (NSC - 1) * R0, LO)])


def _tc_pre0(x_ref, degp_ref, w_ref, g_ref, dinv_ref):
    deg = degp_ref[0, :, 0] + degp_ref[1, :, 0] + 1.0
    dinv = lax.rsqrt(deg)
    g = lax.dot_general(x_ref[...], w_ref[...], (((1,), (1,)), ((), ())),
                        preferred_element_type=jnp.float32)
    g = g * dinv[:, None]
    g_ref[0] = g[:, :H]
    g_ref[1] = g[:, H:]
    dinv_ref[...] = dinv[:, None]


def _tc_post(s_ref, g_ref, dinv_ref, b_ref, t_ref, ps_ref, pq_ref):
    i = pl.program_id(0)
    sv = jnp.concatenate([s_ref[0], s_ref[1]], axis=1)
    gv = jnp.concatenate([g_ref[0], g_ref[1]], axis=1)
    t = (sv + gv) * dinv_ref[...] + b_ref[...]
    rows = lax.broadcasted_iota(jnp.int32, (RB, 1), 0) + i * RB
    t = jnp.where(rows < N, t, 0.0)
    t_ref[...] = t
    ps_ref[...] = jnp.sum(t, axis=0, keepdims=True)[None]
    pq_ref[...] = jnp.sum(t * t, axis=0, keepdims=True)[None]


def _bn_coeffs(ps_ref, pq_ref, gam_ref, bet_ref):
    mu = jnp.sum(ps_ref[...], axis=0) / N
    ex2 = jnp.sum(pq_ref[...], axis=0) / N
    var = ex2 - mu * mu
    a = gam_ref[...] * lax.rsqrt(var + 1e-5)
    return a, bet_ref[...] - mu * a


def _tc_fuse(t_ref, ps_ref, pq_ref, gam_ref, bet_ref, dinv_ref, w_ref, g_ref):
    a, cshift = _bn_coeffs(ps_ref, pq_ref, gam_ref, bet_ref)
    z = t_ref[...] * a + cshift
    z = jnp.where(z >= 0, z, 0.01 * z)
    g = lax.dot_general(z, w_ref[...], (((1,), (1,)), ((), ())),
                        preferred_element_type=jnp.float32)
    g = g * dinv_ref[...]
    g_ref[0] = g[:, :H]
    g_ref[1] = g[:, H:]


def _tc_final(t_ref, ps_ref, pq_ref, gam_ref, bet_ref, o_ref):
    a, cshift = _bn_coeffs(ps_ref, pq_ref, gam_ref, bet_ref)
    o_ref[...] = t_ref[...] * a + cshift


_f32 = jnp.float32

_pre0_call = pl.pallas_call(
    _tc_pre0,
    grid=(NRB,),
    in_specs=[
        pl.BlockSpec((RB, D), lambda i: (i, 0)),
        pl.BlockSpec((2, RB, H), lambda i: (0, i, 0)),
        pl.BlockSpec((D, D), lambda i: (0, 0)),
    ],
    out_specs=[
        pl.BlockSpec((2, RB, H), lambda i: (0, i, 0)),
        pl.BlockSpec((RB, 1), lambda i: (i, 0)),
    ],
    out_shape=[
        jax.ShapeDtypeStruct((2, N, H), _f32),
        jax.ShapeDtypeStruct((N, 1), _f32),
    ],
)

_post_call = pl.pallas_call(
    _tc_post,
    grid=(NRB,),
    in_specs=[
        pl.BlockSpec((2, RB, H), lambda i: (0, i, 0)),
        pl.BlockSpec((2, RB, H), lambda i: (0, i, 0)),
        pl.BlockSpec((RB, 1), lambda i: (i, 0)),
        pl.BlockSpec((1, D), lambda i: (0, 0)),
    ],
    out_specs=[
        pl.BlockSpec((RB, D), lambda i: (i, 0)),
        pl.BlockSpec((1, 1, D), lambda i: (i, 0, 0)),
        pl.BlockSpec((1, 1, D), lambda i: (i, 0, 0)),
    ],
    out_shape=[
        jax.ShapeDtypeStruct((N, D), _f32),
        jax.ShapeDtypeStruct((NRB, 1, D), _f32),
        jax.ShapeDtypeStruct((NRB, 1, D), _f32),
    ],
)

_fuse_call = pl.pallas_call(
    _tc_fuse,
    grid=(NRB,),
    in_specs=[
        pl.BlockSpec((RB, D), lambda i: (i, 0)),
        pl.BlockSpec((NRB, 1, D), lambda i: (0, 0, 0)),
        pl.BlockSpec((NRB, 1, D), lambda i: (0, 0, 0)),
        pl.BlockSpec((1, D), lambda i: (0, 0)),
        pl.BlockSpec((1, D), lambda i: (0, 0)),
        pl.BlockSpec((RB, 1), lambda i: (i, 0)),
        pl.BlockSpec((D, D), lambda i: (0, 0)),
    ],
    out_specs=[pl.BlockSpec((2, RB, H), lambda i: (0, i, 0))],
    out_shape=[jax.ShapeDtypeStruct((2, N, H), _f32)],
)

_final_call = pl.pallas_call(
    _tc_final,
    grid=(NRB,),
    in_specs=[
        pl.BlockSpec((RB, D), lambda i: (i, 0)),
        pl.BlockSpec((NRB, 1, D), lambda i: (0, 0, 0)),
        pl.BlockSpec((NRB, 1, D), lambda i: (0, 0, 0)),
        pl.BlockSpec((1, D), lambda i: (0, 0)),
        pl.BlockSpec((1, D), lambda i: (0, 0)),
    ],
    out_specs=pl.BlockSpec((RB, D), lambda i: (i, 0)),
    out_shape=jax.ShapeDtypeStruct((N, D), _f32),
)


def kernel(x, edge_index, W0, b0, g0, be0, W1, b1, g1, be1, W2, b2, g2, be2):
    src = edge_index[0].astype(jnp.int32)
    dst = edge_index[1].astype(jnp.int32)
    e = src.shape[0]
    padn = NSC * NBAT * BS - e
    srcp = jnp.concatenate([src, jnp.zeros((padn,), jnp.int32)])
    srcp = srcp.reshape(NSC, NBAT, BS)
    dstps = jnp.concatenate([dst, jnp.full((padn,), TRASH, jnp.int32)])
    dstps = dstps.reshape(NSC * NCHUNK, CH, BS)
    srcp2 = jnp.concatenate([srcp, srcp + N], axis=0)
    srcp2 = srcp2.reshape(2 * NSC * NCHUNK, CH, BS)
    padd = NSC * EPW - e
    dstp = jnp.concatenate([dst, jnp.full((padd,), TRASH, jnp.int32)])
    dstp = dstp.reshape(NSC, NB, B)

    zrows = jnp.zeros((LZ, H), _f32)

    degp = _sc_deg(dstp, jnp.ones((B, H), _f32), zrows).reshape(2, N, H)

    ga, dinv = _pre0_call(x, degp, W0)
    sa = _sc_segsum(ga.reshape(2 * N, H), srcp2, dstps, zrows).reshape(2, N, H)
    t, ps, pq = _post_call(sa, ga, dinv, b0.reshape(1, D))

    ga = _fuse_call(t, ps, pq, g0.reshape(1, D), be0.reshape(1, D), dinv, W1)[0]
    sa = _sc_segsum(ga.reshape(2 * N, H), srcp2, dstps, zrows).reshape(2, N, H)
    t, ps, pq = _post_call(sa, ga, dinv, b1.reshape(1, D))

    ga = _fuse_call(t, ps, pq, g1.reshape(1, D), be1.reshape(1, D), dinv, W2)[0]
    sa = _sc_segsum(ga.reshape(2 * N, H), srcp2, dstps, zrows).reshape(2, N, H)
    t, ps, pq = _post_call(sa, ga, dinv, b2.reshape(1, D))

    return _final_call(t, ps, pq, g2.reshape(1, D), be2.reshape(1, D))


# speedup vs baseline: 11.1218x; 1.0871x over previous
import functools

import jax
import jax.numpy as jnp
from jax import lax
from jax.experimental import pallas as pl
from jax.experimental.pallas import tpu as pltpu
from jax.experimental.pallas import tpu_sc as plsc

N = 10000
D = 256
H = 128
NSC = 16
B = 128
NB = 80
EPW = NB * B
ACC_ROWS = 10016
R0 = 624
LZ = 656
LO = 640
TRASH = N
BS = 120
NBAT = 84
CH = 6
NCHUNK = NBAT // CH
RB = 512
NRB = 20

_mesh = plsc.VectorSubcoreMesh(core_axis_name="c", subcore_axis_name="s")


@functools.partial(
    pl.kernel,
    out_type=jax.ShapeDtypeStruct((2 * N, H), jnp.float32),
    mesh=_mesh,
    scratch_types=[
        pltpu.VMEM_SHARED((ACC_ROWS, H), jnp.float32),
        pltpu.VMEM((CH, BS), jnp.int32),
        pltpu.VMEM((CH, BS), jnp.int32),
        pltpu.VMEM((CH, BS), jnp.int32),
        pltpu.VMEM((CH, BS), jnp.int32),
        pltpu.VMEM((BS, H), jnp.float32),
        pltpu.VMEM((BS, H), jnp.float32),
        pltpu.VMEM((BS, H), jnp.float32),
        pltpu.SemaphoreType.DMA,
        pltpu.SemaphoreType.DMA,
        pltpu.SemaphoreType.DMA,
        pltpu.SemaphoreType.DMA,
        pltpu.SemaphoreType.DMA,
        pltpu.SemaphoreType.DMA,
        pltpu.SemaphoreType.DMA,
    ],
)
def _sc_segsum(g2, srcp2, dstp2, zrows, s_out, acc, sv0, dv0, sv1, dv1,
               buf0, buf1, buf2, sg0, sg1, sg2, ss0, ss1, ss2, si):
    c = lax.axis_index("c")
    s = lax.axis_index("s")
    w = c * NSC + s
    bufs = (buf0, buf1, buf2)
    sg = (sg0, sg1, sg2)
    ss = (ss0, ss1, ss2)
    svs = (sv0, sv1)
    dvs = (dv0, dv1)

    pltpu.sync_copy(srcp2.at[w * NCHUNK], sv0)
    pltpu.sync_copy(dstp2.at[s * NCHUNK], dv0)

    @pl.when(s < NSC - 1)
    def _():
        pltpu.sync_copy(zrows.at[pl.ds(0, R0)], acc.at[pl.ds(s * R0, R0)])

    @pl.when(s == NSC - 1)
    def _():
        pltpu.sync_copy(zrows, acc.at[pl.ds((NSC - 1) * R0, LZ)])

    pltpu.async_copy(g2.at[sv0.at[0]], buf0, sg0)
    pltpu.async_copy(g2.at[sv0.at[1]], buf1, sg1)
    plsc.subcore_barrier()

    def _chunk(kk, p):
        sv, dv = svs[p], dvs[p]
        svq, dvq = svs[1 - p], dvs[1 - p]
        for t in range(CH):
            b = t % 3
            bn = (t + 2) % 3
            pltpu.make_async_copy(g2.at[sv.at[t]], bufs[b], sg[b]).wait()
            pltpu.async_copy(bufs[b], acc.at[dv.at[t]], ss[b], add=True)

            if t == 2:
                @pl.when(kk < NCHUNK - 1)
                def _():
                    pltpu.async_copy(srcp2.at[w * NCHUNK + kk + 1], svq, si)
                    pltpu.async_copy(dstp2.at[s * NCHUNK + kk + 1], dvq, si)

            def _wait_prev(t=t, bn=bn):
                row = t - 1 if t >= 1 else CH - 1
                dref = dv if t >= 1 else dvq
                pltpu.make_async_copy(bufs[bn], acc.at[dref.at[row]],
                                      ss[bn]).wait()

            if t < CH - 2:
                if t == 0:
                    @pl.when(kk > 0)
                    def _():
                        _wait_prev()
                else:
                    _wait_prev()
                pltpu.async_copy(g2.at[sv.at[t + 2]], bufs[bn], sg[bn])
            else:
                @pl.when(kk < NCHUNK - 1)
                def _():
                    _wait_prev()
                    if t == CH - 2:
                        pltpu.make_async_copy(
                            srcp2.at[w * NCHUNK + kk + 1], svq, si).wait()
                        pltpu.make_async_copy(
                            dstp2.at[s * NCHUNK + kk + 1], dvq, si).wait()
                    pltpu.async_copy(g2.at[svq.at[t + 2 - CH]], bufs[bn],
                                     sg[bn])

    @pl.loop(0, NCHUNK)
    def _(kk):
        @pl.when(kk % 2 == 0)
        def _():
            _chunk(kk, 0)

        @pl.when(kk % 2 == 1)
        def _():
            _chunk(kk, 1)

    lastp = (NCHUNK - 1) % 2
    for t in range(CH - 3, CH):
        b = t % 3
        pltpu.make_async_copy(bufs[b], acc.at[dvs[lastp].at[t]],
                              ss[b]).wait()
    plsc.subcore_barrier()

    @pl.when(s < NSC - 1)
    def _():
        pltpu.sync_copy(acc.at[pl.ds(s * R0, R0)],
                        s_out.at[pl.ds(c * N + s * R0, R0)])

    @pl.when(s == NSC - 1)
    def _():
        pltpu.sync_copy(acc.at[pl.ds((NSC - 1) * R0, LO)],
                        s_out.at[pl.ds(c * N + (NSC - 1) * R0, LO)])


@functools.partial(
    pl.kernel,
    out_type=jax.ShapeDtypeStruct((2 * N, H), jnp.float32),
    mesh=_mesh,
    scratch_types=[
        pltpu.VMEM_SHARED((ACC_ROWS, H), jnp.float32),
        pltpu.VMEM((NB, B), jnp.int32),
        pltpu.VMEM((B, H), jnp.float32),
        pltpu.SemaphoreType.DMA,
    ],
)
def _sc_deg(dstp, ones_hbm, zrows, deg_out, acc, dstv, onesv, ssem):
    c = lax.axis_index("c")
    s = lax.axis_index("s")
    pltpu.sync_copy(dstp.at[s], dstv)
    pltpu.sync_copy(ones_hbm, onesv)

    @pl.when(s < NSC - 1)
    def _():
        pltpu.sync_copy(zrows.at[pl.ds(0, R0)], acc.at[pl.ds(s * R0, R0)])

    @pl.when(s == NSC - 1)
    def _():
        pltpu.sync_copy(zrows, acc.at[pl.ds((NSC - 1) * R0, LZ)])

    plsc.subcore_barrier()

    @pl.loop(0, NB)
    def _(j):
        @pl.when(j % 2 == c)
        def _():
            pltpu.async_copy(onesv, acc.at[dstv.at[j]], ssem, add=True)

    @pl.loop(0, NB)
    def _(j):
        @pl.when(j % 2 == c)
        def _():
            pltpu.make_async_copy(onesv, acc.at[dstv.at[0]], ssem).wait()

    plsc.subcore_barrier()

    @pl.when(s < NSC - 1)
    def _():
        pltpu.sync_copy(acc.at[pl.ds(s * R0, R0)],
                        deg_out.at[pl.ds(c * N + s * R0, R0)])

    @pl.when(s == NSC - 1)
    def _():
        pltpu.sync_copy(acc.at[pl.ds((NSC - 1) * R0, LO)],
                        deg_out.at[pl.ds(c * N + (NSC - 1) * R0, LO)])


def _tc_pre0(x_ref, degp_ref, w_ref, g_ref, dinv_ref):
    deg = degp_ref[0, :, 0] + degp_ref[1, :, 0] + 1.0
    dinv = lax.rsqrt(deg)
    g = lax.dot_general(x_ref[...], w_ref[...], (((1,), (1,)), ((), ())),
                        preferred_element_type=jnp.float32)
    g = g * dinv[:, None]
    g_ref[0] = g[:, :H]
    g_ref[1] = g[:, H:]
    dinv_ref[...] = dinv[:, None]


def _tc_post(s_ref, g_ref, dinv_ref, b_ref, t_ref, ps_ref, pq_ref):
    i = pl.program_id(0)
    sv = jnp.concatenate([s_ref[0], s_ref[1]], axis=1)
    gv = jnp.concatenate([g_ref[0], g_ref[1]], axis=1)
    t = (sv + gv) * dinv_ref[...] + b_ref[...]
    rows = lax.broadcasted_iota(jnp.int32, (RB, 1), 0) + i * RB
    t = jnp.where(rows < N, t, 0.0)
    t_ref[...] = t
    ps_ref[...] = jnp.sum(t, axis=0, keepdims=True)[None]
    pq_ref[...] = jnp.sum(t * t, axis=0, keepdims=True)[None]


def _bn_coeffs(ps_ref, pq_ref, gam_ref, bet_ref):
    mu = jnp.sum(ps_ref[...], axis=0) / N
    ex2 = jnp.sum(pq_ref[...], axis=0) / N
    var = ex2 - mu * mu
    a = gam_ref[...] * lax.rsqrt(var + 1e-5)
    return a, bet_ref[...] - mu * a


def _tc_fuse(t_ref, ps_ref, pq_ref, gam_ref, bet_ref, dinv_ref, w_ref, g_ref):
    a, cshift = _bn_coeffs(ps_ref, pq_ref, gam_ref, bet_ref)
    z = t_ref[...] * a + cshift
    z = jnp.where(z >= 0, z, 0.01 * z)
    g = lax.dot_general(z, w_ref[...], (((1,), (1,)), ((), ())),
                        preferred_element_type=jnp.float32)
    g = g * dinv_ref[...]
    g_ref[0] = g[:, :H]
    g_ref[1] = g[:, H:]


def _tc_final(t_ref, ps_ref, pq_ref, gam_ref, bet_ref, o_ref):
    a, cshift = _bn_coeffs(ps_ref, pq_ref, gam_ref, bet_ref)
    o_ref[...] = t_ref[...] * a + cshift


_f32 = jnp.float32

_pre0_call = pl.pallas_call(
    _tc_pre0,
    grid=(NRB,),
    in_specs=[
        pl.BlockSpec((RB, D), lambda i: (i, 0)),
        pl.BlockSpec((2, RB, H), lambda i: (0, i, 0)),
        pl.BlockSpec((D, D), lambda i: (0, 0)),
    ],
    out_specs=[
        pl.BlockSpec((2, RB, H), lambda i: (0, i, 0)),
        pl.BlockSpec((RB, 1), lambda i: (i, 0)),
    ],
    out_shape=[
        jax.ShapeDtypeStruct((2, N, H), _f32),
        jax.ShapeDtypeStruct((N, 1), _f32),
    ],
)

_post_call = pl.pallas_call(
    _tc_post,
    grid=(NRB,),
    in_specs=[
        pl.BlockSpec((2, RB, H), lambda i: (0, i, 0)),
        pl.BlockSpec((2, RB, H), lambda i: (0, i, 0)),
        pl.BlockSpec((RB, 1), lambda i: (i, 0)),
        pl.BlockSpec((1, D), lambda i: (0, 0)),
    ],
    out_specs=[
        pl.BlockSpec((RB, D), lambda i: (i, 0)),
        pl.BlockSpec((1, 1, D), lambda i: (i, 0, 0)),
        pl.BlockSpec((1, 1, D), lambda i: (i, 0, 0)),
    ],
    out_shape=[
        jax.ShapeDtypeStruct((N, D), _f32),
        jax.ShapeDtypeStruct((NRB, 1, D), _f32),
        jax.ShapeDtypeStruct((NRB, 1, D), _f32),
    ],
)

_fuse_call = pl.pallas_call(
    _tc_fuse,
    grid=(NRB,),
    in_specs=[
        pl.BlockSpec((RB, D), lambda i: (i, 0)),
        pl.BlockSpec((NRB, 1, D), lambda i: (0, 0, 0)),
        pl.BlockSpec((NRB, 1, D), lambda i: (0, 0, 0)),
        pl.BlockSpec((1, D), lambda i: (0, 0)),
        pl.BlockSpec((1, D), lambda i: (0, 0)),
        pl.BlockSpec((RB, 1), lambda i: (i, 0)),
        pl.BlockSpec((D, D), lambda i: (0, 0)),
    ],
    out_specs=[pl.BlockSpec((2, RB, H), lambda i: (0, i, 0))],
    out_shape=[jax.ShapeDtypeStruct((2, N, H), _f32)],
)

_final_call = pl.pallas_call(
    _tc_final,
    grid=(NRB,),
    in_specs=[
        pl.BlockSpec((RB, D), lambda i: (i, 0)),
        pl.BlockSpec((NRB, 1, D), lambda i: (0, 0, 0)),
        pl.BlockSpec((NRB, 1, D), lambda i: (0, 0, 0)),
        pl.BlockSpec((1, D), lambda i: (0, 0)),
        pl.BlockSpec((1, D), lambda i: (0, 0)),
    ],
    out_specs=pl.BlockSpec((RB, D), lambda i: (i, 0)),
    out_shape=jax.ShapeDtypeStruct((N, D), _f32),
)


def kernel(x, edge_index, W0, b0, g0, be0, W1, b1, g1, be1, W2, b2, g2, be2):
    src = edge_index[0].astype(jnp.int32)
    dst = edge_index[1].astype(jnp.int32)
    e = src.shape[0]
    padn = NSC * NBAT * BS - e
    srcp = jnp.concatenate([src, jnp.zeros((padn,), jnp.int32)])
    srcp = srcp.reshape(NSC, NBAT, BS)
    dstps = jnp.concatenate([dst, jnp.full((padn,), TRASH, jnp.int32)])
    dstps = dstps.reshape(NSC * NCHUNK, CH, BS)
    srcp2 = jnp.concatenate([srcp, srcp + N], axis=0)
    srcp2 = srcp2.reshape(2 * NSC * NCHUNK, CH, BS)
    padd = NSC * EPW - e
    dstp = jnp.concatenate([dst, jnp.full((padd,), TRASH, jnp.int32)])
    dstp = dstp.reshape(NSC, NB, B)

    zrows = jnp.zeros((LZ, H), _f32)

    degp = _sc_deg(dstp, jnp.ones((B, H), _f32), zrows).reshape(2, N, H)

    ga, dinv = _pre0_call(x, degp, W0)
    sa = _sc_segsum(ga.reshape(2 * N, H), srcp2, dstps, zrows).reshape(2, N, H)
    t, ps, pq = _post_call(sa, ga, dinv, b0.reshape(1, D))

    ga = _fuse_call(t, ps, pq, g0.reshape(1, D), be0.reshape(1, D), dinv, W1)[0]
    sa = _sc_segsum(ga.reshape(2 * N, H), srcp2, dstps, zrows).reshape(2, N, H)
    t, ps, pq = _post_call(sa, ga, dinv, b1.reshape(1, D))

    ga = _fuse_call(t, ps, pq, g1.reshape(1, D), be1.reshape(1, D), dinv, W2)[0]
    sa = _sc_segsum(ga.reshape(2 * N, H), srcp2, dstps, zrows).reshape(2, N, H)
    t, ps, pq = _post_call(sa, ga, dinv, b2.reshape(1, D))

    return _final_call(t, ps, pq, g2.reshape(1, D), be2.reshape(1, D))


# speedup vs baseline: 11.9531x; 1.0747x over previous
import functools

import jax
import jax.numpy as jnp
from jax import lax
from jax.experimental import pallas as pl
from jax.experimental.pallas import tpu as pltpu
from jax.experimental.pallas import tpu_sc as plsc

N = 10000
D = 256
H = 128
NSC = 16
B = 128
NB = 80
EPW = NB * B
ACC_ROWS = 10008
R0 = 624
LZ = 648
LO = 640
TRASH = N
BS = 120
NBAT = 84
CH = 6
NCHUNK = NBAT // CH
RB = 1024
NRB = 10

_mesh = plsc.VectorSubcoreMesh(core_axis_name="c", subcore_axis_name="s")


@functools.partial(
    pl.kernel,
    out_type=jax.ShapeDtypeStruct((2 * N, H), jnp.float32),
    mesh=_mesh,
    scratch_types=[
        pltpu.VMEM_SHARED((ACC_ROWS, H), jnp.float32),
        pltpu.VMEM((CH, BS), jnp.int32),
        pltpu.VMEM((CH, BS), jnp.int32),
        pltpu.VMEM((CH, BS), jnp.int32),
        pltpu.VMEM((CH, BS), jnp.int32),
        pltpu.VMEM((BS, H), jnp.float32),
        pltpu.VMEM((BS, H), jnp.float32),
        pltpu.VMEM((BS, H), jnp.float32),
        pltpu.SemaphoreType.DMA,
        pltpu.SemaphoreType.DMA,
        pltpu.SemaphoreType.DMA,
        pltpu.SemaphoreType.DMA,
        pltpu.SemaphoreType.DMA,
        pltpu.SemaphoreType.DMA,
        pltpu.SemaphoreType.DMA,
    ],
)
def _sc_segsum(g2, srcp2, dstp2, zrows, s_out, acc, sv0, dv0, sv1, dv1,
               buf0, buf1, buf2, sg0, sg1, sg2, ss0, ss1, ss2, si):
    c = lax.axis_index("c")
    s = lax.axis_index("s")
    w = c * NSC + s
    bufs = (buf0, buf1, buf2)
    sg = (sg0, sg1, sg2)
    ss = (ss0, ss1, ss2)
    svs = (sv0, sv1)
    dvs = (dv0, dv1)

    pltpu.sync_copy(srcp2.at[w * NCHUNK], sv0)
    pltpu.sync_copy(dstp2.at[s * NCHUNK], dv0)

    @pl.when(s < NSC - 1)
    def _():
        pltpu.sync_copy(zrows.at[pl.ds(0, R0)], acc.at[pl.ds(s * R0, R0)])

    @pl.when(s == NSC - 1)
    def _():
        pltpu.sync_copy(zrows, acc.at[pl.ds((NSC - 1) * R0, LZ)])

    pltpu.async_copy(g2.at[sv0.at[0]], buf0, sg0)
    pltpu.async_copy(g2.at[sv0.at[1]], buf1, sg1)
    plsc.subcore_barrier()

    def _chunk(kk, p):
        sv, dv = svs[p], dvs[p]
        svq, dvq = svs[1 - p], dvs[1 - p]
        for t in range(CH):
            b = t % 3
            bn = (t + 2) % 3
            pltpu.make_async_copy(g2.at[sv.at[t]], bufs[b], sg[b]).wait()
            pltpu.async_copy(bufs[b], acc.at[dv.at[t]], ss[b], add=True)

            if t == 2:
                @pl.when(kk < NCHUNK - 1)
                def _():
                    pltpu.async_copy(srcp2.at[w * NCHUNK + kk + 1], svq, si)
                    pltpu.async_copy(dstp2.at[s * NCHUNK + kk + 1], dvq, si)

            def _wait_prev(t=t, bn=bn):
                row = t - 1 if t >= 1 else CH - 1
                dref = dv if t >= 1 else dvq
                pltpu.make_async_copy(bufs[bn], acc.at[dref.at[row]],
                                      ss[bn]).wait()

            if t < CH - 2:
                if t == 0:
                    @pl.when(kk > 0)
                    def _():
                        _wait_prev()
                else:
                    _wait_prev()
                pltpu.async_copy(g2.at[sv.at[t + 2]], bufs[bn], sg[bn])
            else:
                @pl.when(kk < NCHUNK - 1)
                def _():
                    _wait_prev()
                    if t == CH - 2:
                        pltpu.make_async_copy(
                            srcp2.at[w * NCHUNK + kk + 1], svq, si).wait()
                        pltpu.make_async_copy(
                            dstp2.at[s * NCHUNK + kk + 1], dvq, si).wait()
                    pltpu.async_copy(g2.at[svq.at[t + 2 - CH]], bufs[bn],
                                     sg[bn])

    @pl.loop(0, NCHUNK)
    def _(kk):
        @pl.when(kk % 2 == 0)
        def _():
            _chunk(kk, 0)

        @pl.when(kk % 2 == 1)
        def _():
            _chunk(kk, 1)

    lastp = (NCHUNK - 1) % 2
    for t in range(CH - 3, CH):
        b = t % 3
        pltpu.make_async_copy(bufs[b], acc.at[dvs[lastp].at[t]],
                              ss[b]).wait()
    plsc.subcore_barrier()

    @pl.when(s < NSC - 1)
    def _():
        pltpu.sync_copy(acc.at[pl.ds(s * R0, R0)],
                        s_out.at[pl.ds(c * N + s * R0, R0)])

    @pl.when(s == NSC - 1)
    def _():
        pltpu.sync_copy(acc.at[pl.ds((NSC - 1) * R0, LO)],
                        s_out.at[pl.ds(c * N + (NSC - 1) * R0, LO)])


@functools.partial(
    pl.kernel,
    out_type=jax.ShapeDtypeStruct((2 * N, H), jnp.float32),
    mesh=_mesh,
    scratch_types=[
        pltpu.VMEM_SHARED((ACC_ROWS, H), jnp.float32),
        pltpu.VMEM((NB, B), jnp.int32),
        pltpu.VMEM((B, H), jnp.float32),
        pltpu.SemaphoreType.DMA,
    ],
)
def _sc_deg(dstp, ones_hbm, zrows, deg_out, acc, dstv, onesv, ssem):
    c = lax.axis_index("c")
    s = lax.axis_index("s")
    pltpu.sync_copy(dstp.at[s], dstv)
    pltpu.sync_copy(ones_hbm, onesv)

    @pl.when(s < NSC - 1)
    def _():
        pltpu.sync_copy(zrows.at[pl.ds(0, R0)], acc.at[pl.ds(s * R0, R0)])

    @pl.when(s == NSC - 1)
    def _():
        pltpu.sync_copy(zrows, acc.at[pl.ds((NSC - 1) * R0, LZ)])

    plsc.subcore_barrier()

    @pl.loop(0, NB)
    def _(j):
        @pl.when(j % 2 == c)
        def _():
            pltpu.async_copy(onesv, acc.at[dstv.at[j]], ssem, add=True)

    @pl.loop(0, NB)
    def _(j):
        @pl.when(j % 2 == c)
        def _():
            pltpu.make_async_copy(onesv, acc.at[dstv.at[0]], ssem).wait()

    plsc.subcore_barrier()

    @pl.when(s < NSC - 1)
    def _():
        pltpu.sync_copy(acc.at[pl.ds(s * R0, R0)],
                        deg_out.at[pl.ds(c * N + s * R0, R0)])

    @pl.when(s == NSC - 1)
    def _():
        pltpu.sync_copy(acc.at[pl.ds((NSC - 1) * R0, LO)],
                        deg_out.at[pl.ds(c * N + (NSC - 1) * R0, LO)])


def _tc_mm0(x_ref, w_ref, h_ref):
    h = lax.dot_general(x_ref[...], w_ref[...], (((1,), (1,)), ((), ())),
                        preferred_element_type=jnp.float32)
    h_ref[0] = h[:, :H]
    h_ref[1] = h[:, H:]


def _tc_scale0(h_ref, degp_ref, g_ref, dinv_ref):
    deg = degp_ref[0, :, 0] + degp_ref[1, :, 0] + 1.0
    dinv = lax.rsqrt(deg)
    g_ref[0] = h_ref[0] * dinv[:, None]
    g_ref[1] = h_ref[1] * dinv[:, None]
    dinv_ref[...] = dinv[:, None]


def _tc_post(s_ref, g_ref, dinv_ref, b_ref, t_ref, ps_ref, pq_ref):
    i = pl.program_id(0)
    sv = jnp.concatenate([s_ref[0], s_ref[1]], axis=1)
    gv = jnp.concatenate([g_ref[0], g_ref[1]], axis=1)
    t = (sv + gv) * dinv_ref[...] + b_ref[...]
    rows = lax.broadcasted_iota(jnp.int32, (RB, 1), 0) + i * RB
    t = jnp.where(rows < N, t, 0.0)
    t_ref[...] = t
    ps_ref[...] = jnp.sum(t, axis=0, keepdims=True)[None]
    pq_ref[...] = jnp.sum(t * t, axis=0, keepdims=True)[None]


def _bn_coeffs(ps_ref, pq_ref, gam_ref, bet_ref):
    mu = jnp.sum(ps_ref[...], axis=0) / N
    ex2 = jnp.sum(pq_ref[...], axis=0) / N
    var = ex2 - mu * mu
    a = gam_ref[...] * lax.rsqrt(var + 1e-5)
    return a, bet_ref[...] - mu * a


def _tc_fuse(t_ref, ps_ref, pq_ref, gam_ref, bet_ref, dinv_ref, w_ref, g_ref):
    a, cshift = _bn_coeffs(ps_ref, pq_ref, gam_ref, bet_ref)
    z = t_ref[...] * a + cshift
    z = jnp.where(z >= 0, z, 0.01 * z)
    g = lax.dot_general(z, w_ref[...], (((1,), (1,)), ((), ())),
                        preferred_element_type=jnp.float32)
    g = g * dinv_ref[...]
    g_ref[0] = g[:, :H]
    g_ref[1] = g[:, H:]


def _tc_final(t_ref, ps_ref, pq_ref, gam_ref, bet_ref, o_ref):
    a, cshift = _bn_coeffs(ps_ref, pq_ref, gam_ref, bet_ref)
    o_ref[...] = t_ref[...] * a + cshift


_f32 = jnp.float32

_mm0_call = pl.pallas_call(
    _tc_mm0,
    grid=(NRB,),
    in_specs=[
        pl.BlockSpec((RB, D), lambda i: (i, 0)),
        pl.BlockSpec((D, D), lambda i: (0, 0)),
    ],
    out_specs=[pl.BlockSpec((2, RB, H), lambda i: (0, i, 0))],
    out_shape=[jax.ShapeDtypeStruct((2, N, H), _f32)],
)

_scale0_call = pl.pallas_call(
    _tc_scale0,
    grid=(NRB,),
    in_specs=[
        pl.BlockSpec((2, RB, H), lambda i: (0, i, 0)),
        pl.BlockSpec((2, RB, H), lambda i: (0, i, 0)),
    ],
    out_specs=[
        pl.BlockSpec((2, RB, H), lambda i: (0, i, 0)),
        pl.BlockSpec((RB, 1), lambda i: (i, 0)),
    ],
    out_shape=[
        jax.ShapeDtypeStruct((2, N, H), _f32),
        jax.ShapeDtypeStruct((N, 1), _f32),
    ],
)

_post_call = pl.pallas_call(
    _tc_post,
    grid=(NRB,),
    in_specs=[
        pl.BlockSpec((2, RB, H), lambda i: (0, i, 0)),
        pl.BlockSpec((2, RB, H), lambda i: (0, i, 0)),
        pl.BlockSpec((RB, 1), lambda i: (i, 0)),
        pl.BlockSpec((1, D), lambda i: (0, 0)),
    ],
    out_specs=[
        pl.BlockSpec((RB, D), lambda i: (i, 0)),
        pl.BlockSpec((1, 1, D), lambda i: (i, 0, 0)),
        pl.BlockSpec((1, 1, D), lambda i: (i, 0, 0)),
    ],
    out_shape=[
        jax.ShapeDtypeStruct((N, D), _f32),
        jax.ShapeDtypeStruct((NRB, 1, D), _f32),
        jax.ShapeDtypeStruct((NRB, 1, D), _f32),
    ],
)

_fuse_call = pl.pallas_call(
    _tc_fuse,
    grid=(NRB,),
    in_specs=[
        pl.BlockSpec((RB, D), lambda i: (i, 0)),
        pl.BlockSpec((NRB, 1, D), lambda i: (0, 0, 0)),
        pl.BlockSpec((NRB, 1, D), lambda i: (0, 0, 0)),
        pl.BlockSpec((1, D), lambda i: (0, 0)),
        pl.BlockSpec((1, D), lambda i: (0, 0)),
        pl.BlockSpec((RB, 1), lambda i: (i, 0)),
        pl.BlockSpec((D, D), lambda i: (0, 0)),
    ],
    out_specs=[pl.BlockSpec((2, RB, H), lambda i: (0, i, 0))],
    out_shape=[jax.ShapeDtypeStruct((2, N, H), _f32)],
)

_final_call = pl.pallas_call(
    _tc_final,
    grid=(NRB,),
    in_specs=[
        pl.BlockSpec((RB, D), lambda i: (i, 0)),
        pl.BlockSpec((NRB, 1, D), lambda i: (0, 0, 0)),
        pl.BlockSpec((NRB, 1, D), lambda i: (0, 0, 0)),
        pl.BlockSpec((1, D), lambda i: (0, 0)),
        pl.BlockSpec((1, D), lambda i: (0, 0)),
    ],
    out_specs=pl.BlockSpec((RB, D), lambda i: (i, 0)),
    out_shape=jax.ShapeDtypeStruct((N, D), _f32),
)


def kernel(x, edge_index, W0, b0, g0, be0, W1, b1, g1, be1, W2, b2, g2, be2):
    src = edge_index[0].astype(jnp.int32)
    dst = edge_index[1].astype(jnp.int32)
    e = src.shape[0]
    padn = NSC * NBAT * BS - e
    srcp = jnp.concatenate([src, jnp.zeros((padn,), jnp.int32)])
    srcp = srcp.reshape(NSC, NBAT, BS)
    dstps = jnp.concatenate([dst, jnp.full((padn,), TRASH, jnp.int32)])
    dstps = dstps.reshape(NSC * NCHUNK, CH, BS)
    srcp2 = jnp.concatenate([srcp, srcp + N], axis=0)
    srcp2 = srcp2.reshape(2 * NSC * NCHUNK, CH, BS)
    padd = NSC * EPW - e
    dstp = jnp.concatenate([dst, jnp.full((padd,), TRASH, jnp.int32)])
    dstp = dstp.reshape(NSC, NB, B)

    zrows = jnp.zeros((LZ, H), _f32)

    h2 = _mm0_call(x, W0)[0]
    degp = _sc_deg(dstp, jnp.ones((B, H), _f32), zrows).reshape(2, N, H)

    ga, dinv = _scale0_call(h2, degp)
    sa = _sc_segsum(ga.reshape(2 * N, H), srcp2, dstps, zrows).reshape(2, N, H)
    t, ps, pq = _post_call(sa, ga, dinv, b0.reshape(1, D))

    ga = _fuse_call(t, ps, pq, g0.reshape(1, D), be0.reshape(1, D), dinv, W1)[0]
    sa = _sc_segsum(ga.reshape(2 * N, H), srcp2, dstps, zrows).reshape(2, N, H)
    t, ps, pq = _post_call(sa, ga, dinv, b1.reshape(1, D))

    ga = _fuse_call(t, ps, pq, g1.reshape(1, D), be1.reshape(1, D), dinv, W2)[0]
    sa = _sc_segsum(ga.reshape(2 * N, H), srcp2, dstps, zrows).reshape(2, N, H)
    t, ps, pq = _post_call(sa, ga, dinv, b2.reshape(1, D))

    return _final_call(t, ps, pq, g2.reshape(1, D), be2.reshape(1, D))
